# baseline (speedup 1.0000x reference)
import dataclasses
import functools

import jax
import jax.numpy as jnp
from jax import lax
from jax.experimental import pallas as pl
from jax.experimental.pallas import tpu as pltpu
from jax.experimental.pallas import tpu_sc as plsc

N_NODES = 100000
MAX_DEG = 64
EMBED_DIM = 128
BATCH = 16384

NUM_CORES = 2
NUM_SUBCORES = 16
NUM_WORKERS = NUM_CORES * NUM_SUBCORES
SAMPLES_PER_WORKER = BATCH // NUM_WORKERS
CHUNK = 8
NUM_CHUNKS = SAMPLES_PER_WORKER // CHUNK
ROWS_PER_CHUNK = CHUNK * MAX_DEG
NUM_GROUPS = ROWS_PER_CHUNK // 128
LANES = 16


def _sc_body(nodes_hbm, am_hbm, feat_hbm,
             neigh_hbm, self_hbm, amout_hbm,
             nodes_v, am_v, self_v, emb_v, acc_sh, zeros_v,
             idx_g0, idx_g1, idx_g2, idx_g3,
             seg_g0, seg_g1, seg_g2, seg_g3,
             sem):
  idx_refs = (idx_g0, idx_g1, idx_g2, idx_g3)
  seg_refs = (seg_g0, seg_g1, seg_g2, seg_g3)

  sid = lax.axis_index("s")
  wid = sid * NUM_CORES + lax.axis_index("c")
  base0 = wid * SAMPLES_PER_WORKER
  acc_base = sid * CHUNK

  for g in range(NUM_GROUPS):
    for t in range(128 // LANES):
      sample = (g * 128 + t * LANES) // MAX_DEG
      seg_refs[g][pl.ds(t * LANES, LANES)] = (
          jnp.full((LANES,), sample, jnp.int32) + acc_base)
  for j in range(CHUNK):
    for k in range(EMBED_DIM // LANES):
      zeros_v[j, pl.ds(k * LANES, LANES)] = jnp.zeros((LANES,), jnp.float32)

  @pl.loop(0, NUM_CHUNKS)
  def _(c):
    base = base0 + c * CHUNK

    pltpu.sync_copy(nodes_hbm.at[pl.ds(base, CHUNK)], nodes_v)
    pltpu.sync_copy(am_hbm.at[nodes_v], am_v)
    pltpu.sync_copy(feat_hbm.at[nodes_v], self_v)
    pltpu.sync_copy(zeros_v, acc_sh.at[pl.ds(acc_base, CHUNK)])

    for j in range(CHUNK):
      for k in range(MAX_DEG // LANES):
        a = am_v[j, pl.ds(k * LANES, LANES)]
        m = am_v[j, pl.ds(MAX_DEG + k * LANES, LANES)]
        sel = jnp.where(m != 0, a, jnp.full((LANES,), N_NODES, jnp.int32))
        flat = j * MAX_DEG + k * LANES
        idx_refs[flat // 128][pl.ds(flat % 128, LANES)] = sel

    for g in range(NUM_GROUPS):
      pltpu.sync_copy(feat_hbm.at[idx_refs[g]],
                      emb_v.at[pl.ds(g * 128, 128)])
    for g in range(NUM_GROUPS):
      pltpu.sync_copy(emb_v.at[pl.ds(g * 128, 128)],
                      acc_sh.at[seg_refs[g]], add=True)

    pltpu.sync_copy(acc_sh.at[pl.ds(acc_base, CHUNK)],
                    neigh_hbm.at[pl.ds(base, CHUNK)])
    pltpu.sync_copy(self_v, self_hbm.at[pl.ds(base, CHUNK)])
    pltpu.sync_copy(am_v, amout_hbm.at[pl.ds(base, CHUNK)])


def _sc_aggregate(nodes, adj_mask, feat_pad):
  mesh = plsc.VectorSubcoreMesh(core_axis_name="c", subcore_axis_name="s")
  out_type = (
      jax.ShapeDtypeStruct((BATCH, EMBED_DIM), jnp.float32),
      jax.ShapeDtypeStruct((BATCH, EMBED_DIM), jnp.float32),
      jax.ShapeDtypeStruct((BATCH, 2 * MAX_DEG), jnp.int32),
  )
  scratch = [
      pltpu.VMEM((CHUNK,), jnp.int32),
      pltpu.VMEM((CHUNK, 2 * MAX_DEG), jnp.int32),
      pltpu.VMEM((CHUNK, EMBED_DIM), jnp.float32),
      pltpu.VMEM((ROWS_PER_CHUNK, EMBED_DIM), jnp.float32),
      pltpu.VMEM_SHARED((NUM_SUBCORES * CHUNK, EMBED_DIM), jnp.float32),
      pltpu.VMEM((CHUNK, EMBED_DIM), jnp.float32),
  ]
  scratch += [pltpu.VMEM((128,), jnp.int32) for _ in range(NUM_GROUPS)]
  scratch += [pltpu.VMEM((128,), jnp.int32) for _ in range(NUM_GROUPS)]
  scratch += [pltpu.SemaphoreType.DMA]
  cp = pltpu.CompilerParams()
  if "needs_layout_passes" in pltpu.CompilerParams.__dataclass_fields__:
    cp = dataclasses.replace(cp, needs_layout_passes=False)
  kern = pl.kernel(_sc_body, out_type=out_type, mesh=mesh,
                   scratch_types=scratch, compiler_params=cp)
  return kern(nodes, adj_mask, feat_pad)


def _tc_body(s_ref, n_ref, am_ref, wa_ref, wb_ref, b_ref, o_ref):
  m = am_ref[:, MAX_DEG:].astype(jnp.float32)
  cnt = jnp.maximum(jnp.sum(m, axis=1, keepdims=True), 1.0)
  neigh = n_ref[...] / cnt
  acc = jnp.dot(s_ref[...], wa_ref[...], preferred_element_type=jnp.float32)
  acc = acc + jnp.dot(neigh, wb_ref[...], preferred_element_type=jnp.float32)
  o_ref[...] = jnp.maximum(acc + b_ref[...], 0.0)


def _tc_combine(self_feats, neigh_sum, am_rows, W1, b1):
  blk = 1024
  grid = (BATCH // blk,)
  wa = W1[:EMBED_DIM]
  wb = W1[EMBED_DIM:]
  return pl.pallas_call(
      _tc_body,
      grid=grid,
      in_specs=[
          pl.BlockSpec((blk, EMBED_DIM), lambda i: (i, 0)),
          pl.BlockSpec((blk, EMBED_DIM), lambda i: (i, 0)),
          pl.BlockSpec((blk, 2 * MAX_DEG), lambda i: (i, 0)),
          pl.BlockSpec((EMBED_DIM, EMBED_DIM), lambda i: (0, 0)),
          pl.BlockSpec((EMBED_DIM, EMBED_DIM), lambda i: (0, 0)),
          pl.BlockSpec((1, EMBED_DIM), lambda i: (0, 0)),
      ],
      out_specs=pl.BlockSpec((blk, EMBED_DIM), lambda i: (i, 0)),
      out_shape=jax.ShapeDtypeStruct((BATCH, EMBED_DIM), jnp.float32),
  )(self_feats, neigh_sum, am_rows, wa, wb, b1.reshape(1, EMBED_DIM))


@jax.jit
def kernel(nodes, adj, mask, feat_table, W1, b1):
  feat_pad = jnp.concatenate(
      [feat_table, jnp.zeros((8, EMBED_DIM), jnp.float32)], axis=0)
  adj_mask = jnp.concatenate([adj, mask], axis=1)
  neigh_sum, self_feats, am_rows = _sc_aggregate(nodes, adj_mask, feat_pad)
  return _tc_combine(self_feats, neigh_sum, am_rows, W1, b1)


# speedup vs baseline: 1.0016x; 1.0016x over previous
import dataclasses

import jax
import jax.numpy as jnp
from jax import lax
from jax.experimental import pallas as pl
from jax.experimental.pallas import tpu as pltpu
from jax.experimental.pallas import tpu_sc as plsc

N_NODES = 100000
MAX_DEG = 64
EMBED_DIM = 128
BATCH = 16384

NUM_CORES = 2
NUM_SUBCORES = 16
NUM_WORKERS = NUM_CORES * NUM_SUBCORES
SAMPLES_PER_WORKER = BATCH // NUM_WORKERS
CHUNK = 8
NUM_CHUNKS = SAMPLES_PER_WORKER // CHUNK
ROWS_PER_CHUNK = CHUNK * MAX_DEG
GROUP = 128
NUM_GROUPS = ROWS_PER_CHUNK // GROUP
SAMPLES_PER_GROUP = GROUP // MAX_DEG
LANES = 16
SENTINEL = N_NODES


def _sc_body(nodes_hbm, am_hbm, feat_hbm,
             neigh_hbm, self_hbm, cnts_hbm,
             nodes_v, am_v0, am_v1, self_v0, self_v1, emb_v,
             out_v0, out_v1, cnts_v0, cnts_v1,
             idx_g0, idx_g1, idx_g2, idx_g3,
             sem_am0, sem_am1, sem_g0, sem_g1, sem_g2, sem_g3,
             sem_self, sem_out0, sem_out1):
  am_v = (am_v0, am_v1)
  self_v = (self_v0, self_v1)
  out_v = (out_v0, out_v1)
  cnts_v = (cnts_v0, cnts_v1)
  idx_refs = (idx_g0, idx_g1, idx_g2, idx_g3)
  sem_am = (sem_am0, sem_am1)
  sem_g = (sem_g0, sem_g1, sem_g2, sem_g3)
  sem_out = (sem_out0, sem_out1)

  wid = lax.axis_index("s") * NUM_CORES + lax.axis_index("c")
  base0 = wid * SAMPLES_PER_WORKER

  pltpu.sync_copy(nodes_hbm.at[pl.ds(base0, SAMPLES_PER_WORKER)], nodes_v)

  for b in range(2):
    for j in range(CHUNK):
      for k in range(EMBED_DIM // LANES):
        cnts_v[b][j, pl.ds(k * LANES, LANES)] = jnp.zeros((LANES,), jnp.int32)

  def fire_am(c, b):
    pltpu.async_copy(am_hbm.at[nodes_v.at[pl.ds(c * CHUNK, CHUNK)]],
                     am_v[b], sem_am[b])

  def wait_am(b):
    pltpu.make_async_copy(am_hbm.at[pl.ds(0, CHUNK)], am_v[b],
                          sem_am[b]).wait()

  def drain_out(b):
    pltpu.make_async_copy(out_v[b], neigh_hbm.at[pl.ds(0, CHUNK)],
                          sem_out[b]).wait()
    pltpu.make_async_copy(self_v[b], self_hbm.at[pl.ds(0, CHUNK)],
                          sem_out[b]).wait()
    pltpu.make_async_copy(cnts_v[b], cnts_hbm.at[pl.ds(0, CHUNK)],
                          sem_out[b]).wait()

  def sub_body(c, b, fire_next, drain):
    wait_am(b)
    if fire_next:
      fire_am(c + 1, b ^ 1)
    if drain:
      drain_out(b)

    for j in range(CHUNK):
      cnt16 = None
      for k in range(MAX_DEG // LANES):
        a = am_v[b][j, pl.ds(k * LANES, LANES)]
        m = am_v[b][j, pl.ds(MAX_DEG + k * LANES, LANES)]
        sel = jnp.where(m != 0, a, jnp.full((LANES,), SENTINEL, jnp.int32))
        flat = j * MAX_DEG + k * LANES
        idx_refs[flat // GROUP][pl.ds(flat % GROUP, LANES)] = sel
        cnt16 = m if cnt16 is None else cnt16 + m
      cnts_v[b][j, pl.ds(0, LANES)] = cnt16

    for g in range(NUM_GROUPS):
      pltpu.async_copy(feat_hbm.at[idx_refs[g]],
                       emb_v.at[pl.ds(g * GROUP, GROUP)], sem_g[g])
    pltpu.async_copy(feat_hbm.at[nodes_v.at[pl.ds(c * CHUNK, CHUNK)]],
                     self_v[b], sem_self)

    zeros16 = jnp.zeros((LANES,), jnp.float32)
    for g in range(NUM_GROUPS):
      pltpu.make_async_copy(feat_hbm.at[pl.ds(0, GROUP)],
                            emb_v.at[pl.ds(g * GROUP, GROUP)],
                            sem_g[g]).wait()
      for j in range(g * SAMPLES_PER_GROUP, (g + 1) * SAMPLES_PER_GROUP):

        @pl.loop(0, MAX_DEG, init_carry=(zeros16,) * (EMBED_DIM // LANES),
                 unroll=2)
        def acc(i, carry, _j=j):
          row = _j * MAX_DEG + i
          return tuple(
              carry[k] + emb_v[row, pl.ds(k * LANES, LANES)]
              for k in range(EMBED_DIM // LANES))

        for k in range(EMBED_DIM // LANES):
          out_v[b][j, pl.ds(k * LANES, LANES)] = acc[k]

    pltpu.make_async_copy(feat_hbm.at[pl.ds(0, CHUNK)], self_v[b],
                          sem_self).wait()

    base = base0 + c * CHUNK
    pltpu.async_copy(out_v[b], neigh_hbm.at[pl.ds(base, CHUNK)], sem_out[b])
    pltpu.async_copy(self_v[b], self_hbm.at[pl.ds(base, CHUNK)], sem_out[b])
    pltpu.async_copy(cnts_v[b], cnts_hbm.at[pl.ds(base, CHUNK)], sem_out[b])

  fire_am(0, 0)
  sub_body(0, 0, fire_next=True, drain=False)
  sub_body(1, 1, fire_next=True, drain=False)

  @pl.loop(2, NUM_CHUNKS - 2, step=2)
  def _(c):
    sub_body(c, 0, fire_next=True, drain=True)
    sub_body(c + 1, 1, fire_next=True, drain=True)

  sub_body(NUM_CHUNKS - 2, 0, fire_next=True, drain=True)
  sub_body(NUM_CHUNKS - 1, 1, fire_next=False, drain=True)
  drain_out(0)
  drain_out(1)


def _sc_aggregate(nodes, adj_mask, feat_pad):
  mesh = plsc.VectorSubcoreMesh(core_axis_name="c", subcore_axis_name="s")
  out_type = (
      jax.ShapeDtypeStruct((BATCH, EMBED_DIM), jnp.float32),
      jax.ShapeDtypeStruct((BATCH, EMBED_DIM), jnp.float32),
      jax.ShapeDtypeStruct((BATCH, EMBED_DIM), jnp.int32),
  )
  scratch = [
      pltpu.VMEM((SAMPLES_PER_WORKER,), jnp.int32),
      pltpu.VMEM((CHUNK, 2 * MAX_DEG), jnp.int32),
      pltpu.VMEM((CHUNK, 2 * MAX_DEG), jnp.int32),
      pltpu.VMEM((CHUNK, EMBED_DIM), jnp.float32),
      pltpu.VMEM((CHUNK, EMBED_DIM), jnp.float32),
      pltpu.VMEM((ROWS_PER_CHUNK, EMBED_DIM), jnp.float32),
      pltpu.VMEM((CHUNK, EMBED_DIM), jnp.float32),
      pltpu.VMEM((CHUNK, EMBED_DIM), jnp.float32),
      pltpu.VMEM((CHUNK, EMBED_DIM), jnp.int32),
      pltpu.VMEM((CHUNK, EMBED_DIM), jnp.int32),
  ]
  scratch += [pltpu.VMEM((GROUP,), jnp.int32) for _ in range(NUM_GROUPS)]
  scratch += [pltpu.SemaphoreType.DMA] * 9
  cp = pltpu.CompilerParams()
  if "needs_layout_passes" in pltpu.CompilerParams.__dataclass_fields__:
    cp = dataclasses.replace(cp, needs_layout_passes=False)
  kern = pl.kernel(_sc_body, out_type=out_type, mesh=mesh,
                   scratch_types=scratch, compiler_params=cp)
  return kern(nodes, adj_mask, feat_pad)


def _tc_body(s_ref, n_ref, c_ref, wa_ref, wb_ref, b_ref, o_ref):
  cnt = jnp.sum(c_ref[...].astype(jnp.float32), axis=1, keepdims=True)
  cnt = jnp.maximum(cnt, 1.0)
  neigh = n_ref[...] / cnt
  acc = jnp.dot(s_ref[...], wa_ref[...], preferred_element_type=jnp.float32)
  acc = acc + jnp.dot(neigh, wb_ref[...], preferred_element_type=jnp.float32)
  o_ref[...] = jnp.maximum(acc + b_ref[...], 0.0)


def _tc_combine(self_feats, neigh_sum, cnts, W1, b1):
  blk = 1024
  grid = (BATCH // blk,)
  wa = W1[:EMBED_DIM]
  wb = W1[EMBED_DIM:]
  return pl.pallas_call(
      _tc_body,
      grid=grid,
      in_specs=[
          pl.BlockSpec((blk, EMBED_DIM), lambda i: (i, 0)),
          pl.BlockSpec((blk, EMBED_DIM), lambda i: (i, 0)),
          pl.BlockSpec((blk, EMBED_DIM), lambda i: (i, 0)),
          pl.BlockSpec((EMBED_DIM, EMBED_DIM), lambda i: (0, 0)),
          pl.BlockSpec((EMBED_DIM, EMBED_DIM), lambda i: (0, 0)),
          pl.BlockSpec((1, EMBED_DIM), lambda i: (0, 0)),
      ],
      out_specs=pl.BlockSpec((blk, EMBED_DIM), lambda i: (i, 0)),
      out_shape=jax.ShapeDtypeStruct((BATCH, EMBED_DIM), jnp.float32),
  )(self_feats, neigh_sum, cnts, wa, wb, b1.reshape(1, EMBED_DIM))


@jax.jit
def kernel(nodes, adj, mask, feat_table, W1, b1):
  feat_pad = jnp.concatenate(
      [feat_table, jnp.zeros((8, EMBED_DIM), jnp.float32)], axis=0)
  adj_mask = jnp.concatenate([adj, mask], axis=1)
  neigh_sum, self_feats, cnts = _sc_aggregate(nodes, adj_mask, feat_pad)
  return _tc_combine(self_feats, neigh_sum, cnts, W1, b1)


# speedup vs baseline: 29.9625x; 29.9132x over previous
import dataclasses

import jax
import jax.numpy as jnp
from jax import lax
from jax.experimental import pallas as pl
from jax.experimental.pallas import tpu as pltpu
from jax.experimental.pallas import tpu_sc as plsc

N_NODES = 100000
MAX_DEG = 64
EMBED_DIM = 128
BATCH = 16384

NUM_CORES = 2
NUM_SUBCORES = 16
NUM_WORKERS = NUM_CORES * NUM_SUBCORES
SAMPLES_PER_WORKER = BATCH // NUM_WORKERS
CHUNK = 8
NUM_CHUNKS = SAMPLES_PER_WORKER // CHUNK
ROWS_PER_CHUNK = CHUNK * MAX_DEG
GROUP = 128
NUM_GROUPS = ROWS_PER_CHUNK // GROUP
SAMPLES_PER_GROUP = GROUP // MAX_DEG
LANES = 16


def _sc_body(nodes_hbm, am_hbm, feat_hbm,
             neigh_hbm, self_hbm, cnts_hbm,
             nodes_v, am_v0, am_v1, self_v0, self_v1, emb_v,
             out_v0, out_v1, cnts_v0, cnts_v1,
             idx_g0, idx_g1, idx_g2, idx_g3,
             sem_am0, sem_am1, sem_g0, sem_g1, sem_g2, sem_g3,
             sem_self, sem_out0, sem_out1):
  am_v = (am_v0, am_v1)
  self_v = (self_v0, self_v1)
  out_v = (out_v0, out_v1)
  cnts_v = (cnts_v0, cnts_v1)
  idx_refs = (idx_g0, idx_g1, idx_g2, idx_g3)
  sem_am = (sem_am0, sem_am1)
  sem_g = (sem_g0, sem_g1, sem_g2, sem_g3)
  sem_out = (sem_out0, sem_out1)

  wid = lax.axis_index("s") * NUM_CORES + lax.axis_index("c")
  base0 = wid * SAMPLES_PER_WORKER

  pltpu.sync_copy(nodes_hbm.at[pl.ds(base0, SAMPLES_PER_WORKER)], nodes_v)

  for b in range(2):
    for j in range(CHUNK):
      for k in range(EMBED_DIM // LANES):
        cnts_v[b][j, pl.ds(k * LANES, LANES)] = jnp.zeros((LANES,), jnp.int32)

  def fire_am(c, b):
    pltpu.async_copy(am_hbm.at[nodes_v.at[pl.ds(c * CHUNK, CHUNK)]],
                     am_v[b], sem_am[b])

  def wait_am(b):
    pltpu.make_async_copy(am_hbm.at[pl.ds(0, CHUNK)], am_v[b],
                          sem_am[b]).wait()

  def drain_out(b):
    pltpu.make_async_copy(out_v[b], neigh_hbm.at[pl.ds(0, CHUNK)],
                          sem_out[b]).wait()
    pltpu.make_async_copy(self_v[b], self_hbm.at[pl.ds(0, CHUNK)],
                          sem_out[b]).wait()
    pltpu.make_async_copy(cnts_v[b], cnts_hbm.at[pl.ds(0, CHUNK)],
                          sem_out[b]).wait()

  def sub_body(c, b, fire_next, drain):
    wait_am(b)
    if fire_next:
      fire_am(c + 1, b ^ 1)
    if drain:
      drain_out(b)

    iota = lax.iota(jnp.int32, LANES)
    scale16 = [None] * CHUNK
    for j in range(CHUNK):
      a_ch = [am_v[b][j, pl.ds(k * LANES, LANES)]
              for k in range(MAX_DEG // LANES)]
      m_ch = [am_v[b][j, pl.ds(MAX_DEG + k * LANES, LANES)]
              for k in range(MAX_DEG // LANES)]
      bcast0 = jnp.cumsum(jnp.where(iota == 0, a_ch[0], 0))
      cnt16 = None
      for k in range(MAX_DEG // LANES):
        sel = jnp.where(m_ch[k] != 0, a_ch[k], bcast0)
        flat = j * MAX_DEG + k * LANES
        idx_refs[flat // GROUP][pl.ds(flat % GROUP, LANES)] = sel
        cnt16 = m_ch[k] if cnt16 is None else cnt16 + m_ch[k]
      cnts_v[b][j, pl.ds(0, LANES)] = cnt16
      tot16 = plsc.cummax(jnp.flip(jnp.cumsum(cnt16), axis=0))
      scale16[j] = jnp.float32(MAX_DEG) - tot16.astype(jnp.float32)

    for g in range(NUM_GROUPS):
      pltpu.async_copy(feat_hbm.at[idx_refs[g]],
                       emb_v.at[pl.ds(g * GROUP, GROUP)], sem_g[g])
    pltpu.async_copy(feat_hbm.at[nodes_v.at[pl.ds(c * CHUNK, CHUNK)]],
                     self_v[b], sem_self)

    zeros16 = jnp.zeros((LANES,), jnp.float32)
    for g in range(NUM_GROUPS):
      pltpu.make_async_copy(feat_hbm.at[pl.ds(0, GROUP)],
                            emb_v.at[pl.ds(g * GROUP, GROUP)],
                            sem_g[g]).wait()
      for j in range(g * SAMPLES_PER_GROUP, (g + 1) * SAMPLES_PER_GROUP):

        @pl.loop(0, MAX_DEG, init_carry=(zeros16,) * (EMBED_DIM // LANES),
                 unroll=2)
        def acc(i, carry, _j=j):
          row = _j * MAX_DEG + i
          return tuple(
              carry[k] + emb_v[row, pl.ds(k * LANES, LANES)]
              for k in range(EMBED_DIM // LANES))

        for k in range(EMBED_DIM // LANES):
          out_v[b][j, pl.ds(k * LANES, LANES)] = (
              acc[k] - scale16[j] * emb_v[j * MAX_DEG, pl.ds(k * LANES, LANES)])

    pltpu.make_async_copy(feat_hbm.at[pl.ds(0, CHUNK)], self_v[b],
                          sem_self).wait()

    base = base0 + c * CHUNK
    pltpu.async_copy(out_v[b], neigh_hbm.at[pl.ds(base, CHUNK)], sem_out[b])
    pltpu.async_copy(self_v[b], self_hbm.at[pl.ds(base, CHUNK)], sem_out[b])
    pltpu.async_copy(cnts_v[b], cnts_hbm.at[pl.ds(base, CHUNK)], sem_out[b])

  fire_am(0, 0)
  sub_body(0, 0, fire_next=True, drain=False)
  sub_body(1, 1, fire_next=True, drain=False)

  @pl.loop(2, NUM_CHUNKS - 2, step=2)
  def _(c):
    sub_body(c, 0, fire_next=True, drain=True)
    sub_body(c + 1, 1, fire_next=True, drain=True)

  sub_body(NUM_CHUNKS - 2, 0, fire_next=True, drain=True)
  sub_body(NUM_CHUNKS - 1, 1, fire_next=False, drain=True)
  drain_out(0)
  drain_out(1)


def _sc_aggregate(nodes, adj_mask, feat_pad):
  mesh = plsc.VectorSubcoreMesh(core_axis_name="c", subcore_axis_name="s")
  out_type = (
      jax.ShapeDtypeStruct((BATCH, EMBED_DIM), jnp.float32),
      jax.ShapeDtypeStruct((BATCH, EMBED_DIM), jnp.float32),
      jax.ShapeDtypeStruct((BATCH, EMBED_DIM), jnp.int32),
  )
  scratch = [
      pltpu.VMEM((SAMPLES_PER_WORKER,), jnp.int32),
      pltpu.VMEM((CHUNK, 2 * MAX_DEG), jnp.int32),
      pltpu.VMEM((CHUNK, 2 * MAX_DEG), jnp.int32),
      pltpu.VMEM((CHUNK, EMBED_DIM), jnp.float32),
      pltpu.VMEM((CHUNK, EMBED_DIM), jnp.float32),
      pltpu.VMEM((ROWS_PER_CHUNK, EMBED_DIM), jnp.float32),
      pltpu.VMEM((CHUNK, EMBED_DIM), jnp.float32),
      pltpu.VMEM((CHUNK, EMBED_DIM), jnp.float32),
      pltpu.VMEM((CHUNK, EMBED_DIM), jnp.int32),
      pltpu.VMEM((CHUNK, EMBED_DIM), jnp.int32),
  ]
  scratch += [pltpu.VMEM((GROUP,), jnp.int32) for _ in range(NUM_GROUPS)]
  scratch += [pltpu.SemaphoreType.DMA] * 9
  cp = pltpu.CompilerParams()
  if "needs_layout_passes" in pltpu.CompilerParams.__dataclass_fields__:
    cp = dataclasses.replace(cp, needs_layout_passes=False)
  kern = pl.kernel(_sc_body, out_type=out_type, mesh=mesh,
                   scratch_types=scratch, compiler_params=cp)
  return kern(nodes, adj_mask, feat_pad)


def _tc_body(s_ref, n_ref, c_ref, wa_ref, wb_ref, b_ref, o_ref):
  cnt = jnp.sum(c_ref[...].astype(jnp.float32), axis=1, keepdims=True)
  cnt = jnp.maximum(cnt, 1.0)
  neigh = n_ref[...] / cnt
  acc = jnp.dot(s_ref[...], wa_ref[...], preferred_element_type=jnp.float32)
  acc = acc + jnp.dot(neigh, wb_ref[...], preferred_element_type=jnp.float32)
  o_ref[...] = jnp.maximum(acc + b_ref[...], 0.0)


def _tc_combine(self_feats, neigh_sum, cnts, W1, b1):
  blk = 1024
  grid = (BATCH // blk,)
  wa = W1[:EMBED_DIM]
  wb = W1[EMBED_DIM:]
  return pl.pallas_call(
      _tc_body,
      grid=grid,
      in_specs=[
          pl.BlockSpec((blk, EMBED_DIM), lambda i: (i, 0)),
          pl.BlockSpec((blk, EMBED_DIM), lambda i: (i, 0)),
          pl.BlockSpec((blk, EMBED_DIM), lambda i: (i, 0)),
          pl.BlockSpec((EMBED_DIM, EMBED_DIM), lambda i: (0, 0)),
          pl.BlockSpec((EMBED_DIM, EMBED_DIM), lambda i: (0, 0)),
          pl.BlockSpec((1, EMBED_DIM), lambda i: (0, 0)),
      ],
      out_specs=pl.BlockSpec((blk, EMBED_DIM), lambda i: (i, 0)),
      out_shape=jax.ShapeDtypeStruct((BATCH, EMBED_DIM), jnp.float32),
  )(self_feats, neigh_sum, cnts, wa, wb, b1.reshape(1, EMBED_DIM))


@jax.jit
def kernel(nodes, adj, mask, feat_table, W1, b1):
  adj_mask = jnp.concatenate([adj, mask], axis=1)
  neigh_sum, self_feats, cnts = _sc_aggregate(nodes, adj_mask, feat_table)
  return _tc_combine(self_feats, neigh_sum, cnts, W1, b1)


# speedup vs baseline: 49.1751x; 1.6412x over previous
import dataclasses

import jax
import jax.numpy as jnp
from jax import lax
from jax.experimental import pallas as pl
from jax.experimental.pallas import tpu as pltpu
from jax.experimental.pallas import tpu_sc as plsc

N_NODES = 100000
MAX_DEG = 64
EMBED_DIM = 128
BATCH = 16384

NUM_CORES = 2
NUM_SUBCORES = 16
NUM_WORKERS = NUM_CORES * NUM_SUBCORES
SAMPLES_PER_WORKER = BATCH // NUM_WORKERS
BLK = 64
NUM_BLK = SAMPLES_PER_WORKER // BLK
LANES = 16
VPB = BLK // LANES


def _sc_body(nodes_hbm, am_hbm, feat_hbm,
             acc_hbm, self_hbm, fn_hbm, cnt_hbm,
             nodes_v, am_v0, am_v1,
             idx_v0, idx_v1, acc_v0, acc_v1,
             self_v0, self_v1, fn_v0, fn_v1, cnt_v0, cnt_v1,
             sem_in0, sem_in1, sem_selfg0, sem_selfg1, sem_selfd0, sem_selfd1,
             sem_fng0, sem_fng1, sem_fnd0, sem_fnd1,
             sem_add0, sem_add1, sem_accd0, sem_accd1, sem_cntd0, sem_cntd1):
  am_v = (am_v0, am_v1)
  idx_v = (idx_v0, idx_v1)
  acc_v = (acc_v0, acc_v1)
  self_v = (self_v0, self_v1)
  fn_v = (fn_v0, fn_v1)
  cnt_v = (cnt_v0, cnt_v1)
  sem_in = (sem_in0, sem_in1)
  sem_selfg = (sem_selfg0, sem_selfg1)
  sem_selfd = (sem_selfd0, sem_selfd1)
  sem_fng = (sem_fng0, sem_fng1)
  sem_fnd = (sem_fnd0, sem_fnd1)
  sem_add = (sem_add0, sem_add1)
  sem_accd = (sem_accd0, sem_accd1)
  sem_cntd = (sem_cntd0, sem_cntd1)

  wid = lax.axis_index("s") * NUM_CORES + lax.axis_index("c")
  base0 = wid * SAMPLES_PER_WORKER

  pltpu.sync_copy(nodes_hbm.at[pl.ds(base0, SAMPLES_PER_WORKER)], nodes_v)

  def fire_in(i, b):
    idxs = nodes_v.at[pl.ds(i * BLK, BLK)]
    pltpu.async_copy(am_hbm.at[idxs], am_v[b], sem_in[b])
    pltpu.async_copy(feat_hbm.at[idxs], self_v[b], sem_selfg[b])

  def wait_in(b):
    pltpu.make_async_copy(am_hbm.at[pl.ds(0, BLK)], am_v[b],
                          sem_in[b]).wait()

  def prep(b):
    iota16 = lax.iota(jnp.int32, LANES)
    zcol = jnp.zeros((LANES,), jnp.int32)
    for v in range(VPB):
      samp = iota16 + (v * LANES)
      a0 = plsc.load_gather(am_v[b], [samp, zcol])

      @pl.loop(0, MAX_DEG, init_carry=(jnp.zeros((LANES,), jnp.int32),))
      def cnt_loop(k, carry, _samp=samp, _a0=a0, _v=v, _b=b):
        (cnt,) = carry
        kk = zcol + k
        a = plsc.load_gather(am_v[_b], [_samp, kk])
        m = plsc.load_gather(am_v[_b], [_samp, kk + MAX_DEG])
        idx_v[_b][pl.ds(k * BLK + _v * LANES, LANES)] = jnp.where(
            m != 0, a, _a0)
        return (cnt + m,)

      cnt_v[b][pl.ds(v * LANES, LANES)] = cnt_loop[0]

  def zero_acc(b):
    zeros16 = jnp.zeros((LANES,), jnp.float32)

    @pl.loop(0, BLK)
    def _(j, _b=b):
      for kk in range(EMBED_DIM // LANES):
        acc_v[_b][j, pl.ds(kk * LANES, LANES)] = zeros16

  def fire_adds(b):
    @pl.loop(0, MAX_DEG)
    def _(k, _b=b):
      pltpu.async_copy(feat_hbm.at[idx_v[_b].at[pl.ds(k * BLK, BLK)]],
                       acc_v[_b], sem_add[_b], add=True)

    pltpu.async_copy(feat_hbm.at[idx_v[b].at[pl.ds(0, BLK)]],
                     fn_v[b], sem_fng[b])

  def wait_adds(b):
    @pl.loop(0, MAX_DEG)
    def _(k, _b=b):
      pltpu.make_async_copy(feat_hbm.at[pl.ds(0, BLK)], acc_v[_b],
                            sem_add[_b]).wait()

  def drain_prev(i, b):
    pb = b ^ 1
    base = base0 + (i - 1) * BLK
    wait_adds(pb)
    pltpu.async_copy(acc_v[pb], acc_hbm.at[pl.ds(base, BLK)], sem_accd[pb])
    pltpu.async_copy(cnt_v[pb], cnt_hbm.at[pl.ds(base, BLK)], sem_cntd[pb])
    pltpu.make_async_copy(feat_hbm.at[pl.ds(0, BLK)], fn_v[pb],
                          sem_fng[pb]).wait()
    pltpu.async_copy(fn_v[pb], fn_hbm.at[pl.ds(base, BLK)], sem_fnd[pb])
    pltpu.make_async_copy(feat_hbm.at[pl.ds(0, BLK)], self_v[pb],
                          sem_selfg[pb]).wait()
    pltpu.async_copy(self_v[pb], self_hbm.at[pl.ds(base, BLK)], sem_selfd[pb])

  def wait_self_drain(b):
    pltpu.make_async_copy(self_v[b], self_hbm.at[pl.ds(0, BLK)],
                          sem_selfd[b]).wait()

  def wait_fn_drain(b):
    pltpu.make_async_copy(fn_v[b], fn_hbm.at[pl.ds(0, BLK)],
                          sem_fnd[b]).wait()

  def wait_acc_drain(b):
    pltpu.make_async_copy(acc_v[b], acc_hbm.at[pl.ds(0, BLK)],
                          sem_accd[b]).wait()

  def wait_cnt_drain(b):
    pltpu.make_async_copy(cnt_v[b], cnt_hbm.at[pl.ds(0, BLK)],
                          sem_cntd[b]).wait()

  fire_in(0, 0)
  for i in range(NUM_BLK):
    b = i & 1
    wait_in(b)
    if i >= 2:
      wait_cnt_drain(b)
    prep(b)
    if i >= 1:
      drain_prev(i, b)
    if i + 1 < NUM_BLK:
      if i >= 1:
        wait_self_drain(b ^ 1)
      fire_in(i + 1, b ^ 1)
    if i >= 2:
      wait_acc_drain(b)
      wait_fn_drain(b)
    zero_acc(b)
    fire_adds(b)
  drain_prev(NUM_BLK, ((NUM_BLK - 1) & 1) ^ 1)
  for b in range(2):
    wait_acc_drain(b)
    wait_cnt_drain(b)
    wait_fn_drain(b)
    wait_self_drain(b)


def _sc_aggregate(nodes, adj_mask, feat_table):
  mesh = plsc.VectorSubcoreMesh(core_axis_name="c", subcore_axis_name="s")
  out_type = (
      jax.ShapeDtypeStruct((BATCH, EMBED_DIM), jnp.float32),
      jax.ShapeDtypeStruct((BATCH, EMBED_DIM), jnp.float32),
      jax.ShapeDtypeStruct((BATCH, EMBED_DIM), jnp.float32),
      jax.ShapeDtypeStruct((BATCH,), jnp.int32),
  )
  scratch = [
      pltpu.VMEM((SAMPLES_PER_WORKER,), jnp.int32),
      pltpu.VMEM((BLK, 2 * MAX_DEG), jnp.int32),
      pltpu.VMEM((BLK, 2 * MAX_DEG), jnp.int32),
      pltpu.VMEM((MAX_DEG * BLK,), jnp.int32),
      pltpu.VMEM((MAX_DEG * BLK,), jnp.int32),
      pltpu.VMEM((BLK, EMBED_DIM), jnp.float32),
      pltpu.VMEM((BLK, EMBED_DIM), jnp.float32),
      pltpu.VMEM((BLK, EMBED_DIM), jnp.float32),
      pltpu.VMEM((BLK, EMBED_DIM), jnp.float32),
      pltpu.VMEM((BLK, EMBED_DIM), jnp.float32),
      pltpu.VMEM((BLK, EMBED_DIM), jnp.float32),
      pltpu.VMEM((BLK,), jnp.int32),
      pltpu.VMEM((BLK,), jnp.int32),
  ]
  scratch += [pltpu.SemaphoreType.DMA] * 16
  cp = pltpu.CompilerParams()
  if "needs_layout_passes" in pltpu.CompilerParams.__dataclass_fields__:
    cp = dataclasses.replace(cp, needs_layout_passes=False)
  kern = pl.kernel(_sc_body, out_type=out_type, mesh=mesh,
                   scratch_types=scratch, compiler_params=cp)
  return kern(nodes, adj_mask, feat_table)


def _tc_body(s_ref, n_ref, f_ref, c_ref, wa_ref, wb_ref, b_ref, o_ref):
  cnt = jnp.maximum(c_ref[...].astype(jnp.float32), 1.0)
  over = jnp.float32(MAX_DEG) - cnt
  neigh = (n_ref[...] - over * f_ref[...]) / cnt
  acc = jnp.dot(s_ref[...], wa_ref[...], preferred_element_type=jnp.float32)
  acc = acc + jnp.dot(neigh, wb_ref[...], preferred_element_type=jnp.float32)
  o_ref[...] = jnp.maximum(acc + b_ref[...], 0.0)


def _tc_combine(self_feats, neigh_sum, first_neigh, cnts, W1, b1):
  blk = 1024
  grid = (BATCH // blk,)
  wa = W1[:EMBED_DIM]
  wb = W1[EMBED_DIM:]
  return pl.pallas_call(
      _tc_body,
      grid=grid,
      in_specs=[
          pl.BlockSpec((blk, EMBED_DIM), lambda i: (i, 0)),
          pl.BlockSpec((blk, EMBED_DIM), lambda i: (i, 0)),
          pl.BlockSpec((blk, EMBED_DIM), lambda i: (i, 0)),
          pl.BlockSpec((blk, 1), lambda i: (i, 0)),
          pl.BlockSpec((EMBED_DIM, EMBED_DIM), lambda i: (0, 0)),
          pl.BlockSpec((EMBED_DIM, EMBED_DIM), lambda i: (0, 0)),
          pl.BlockSpec((1, EMBED_DIM), lambda i: (0, 0)),
      ],
      out_specs=pl.BlockSpec((blk, EMBED_DIM), lambda i: (i, 0)),
      out_shape=jax.ShapeDtypeStruct((BATCH, EMBED_DIM), jnp.float32),
  )(self_feats, neigh_sum, first_neigh, cnts, wa, wb,
    b1.reshape(1, EMBED_DIM))


@jax.jit
def kernel(nodes, adj, mask, feat_table, W1, b1):
  adj_mask = jnp.concatenate([adj, mask], axis=1)
  neigh_sum, self_feats, first_neigh, cnts = _sc_aggregate(
      nodes, adj_mask, feat_table)
  return _tc_combine(self_feats, neigh_sum, first_neigh,
                     cnts.reshape(BATCH, 1), W1, b1)
